# baseline (device time: 69680 ns/iter reference)
import functools

import jax
import jax.numpy as jnp
from jax import lax
from jax.experimental import pallas as pl
from jax.experimental.pallas import tpu as pltpu

N_DEV = 4
SQ = 256
SKV_LOCAL = 4096
HQ = 8
DH = 128
DM = 1024
SCALE = 0.08838834764831843
NEG = -1e9


NSPLIT = 4
CHUNK = SKV_LOCAL // NSPLIT


def _head_copies(k_hbm, v_hbm, kbuf, vbuf, ksem, vsem, head, slot):
    copies = []
    for c in range(NSPLIT):
        sl = pl.ds(c * CHUNK, CHUNK)
        copies.append(pltpu.make_async_copy(
            k_hbm.at[0, sl, head, :], kbuf.at[slot, sl, :], ksem.at[slot, c]
        ))
        copies.append(pltpu.make_async_copy(
            v_hbm.at[0, sl, head, :], vbuf.at[slot, sl, :], vsem.at[slot, c]
        ))
    return copies


def _attn_body(
    x_ref, wq_ref, k_hbm, v_hbm, o_ref, st_ref,
    bias_ref, kbuf, vbuf, ksem, vsem,
):
    h = pl.program_id(0)
    slot = h % 2
    nslot = (h + 1) % 2

    @pl.when(h == 0)
    def _():
        for cp in _head_copies(k_hbm, v_hbm, kbuf, vbuf, ksem, vsem, 0, 0):
            cp.start()
        my = lax.axis_index("i")
        qb = lax.broadcasted_iota(jnp.int32, (SQ, SKV_LOCAL), 0) // 64
        kb = lax.broadcasted_iota(jnp.int32, (SQ, SKV_LOCAL), 1) // 64 + my * 64
        mask = (qb == kb) | (kb == 0) | (((qb + kb) % 3) == 0)
        bias_ref[...] = jnp.where(mask, 0.0, NEG).astype(jnp.float32)

    @pl.when(h + 1 < HQ)
    def _():
        for cp in _head_copies(
            k_hbm, v_hbm, kbuf, vbuf, ksem, vsem, h + 1, nslot
        ):
            cp.start()

    q = jnp.dot(
        x_ref[0].astype(jnp.bfloat16),
        wq_ref[...].astype(jnp.bfloat16),
        preferred_element_type=jnp.float32,
    )
    copies = _head_copies(k_hbm, v_hbm, kbuf, vbuf, ksem, vsem, h, slot)
    for cp in copies[0::2]:
        cp.wait()
    k = kbuf[slot].astype(jnp.bfloat16)
    s = lax.dot_general(
        q.astype(jnp.bfloat16),
        k,
        ((((1,), (1,))), ((), ())),
        preferred_element_type=jnp.float32,
    )
    s = s * SCALE + bias_ref[...]
    m = jnp.max(s, axis=1, keepdims=True)
    w = jnp.exp(s - m)
    lsum = jnp.sum(w, axis=1, keepdims=True)
    for cp in copies[1::2]:
        cp.wait()
    o = lax.dot_general(
        w.astype(jnp.bfloat16),
        vbuf[slot].astype(jnp.bfloat16),
        ((((1,), (0,))), ((), ())),
        preferred_element_type=jnp.float32,
    )
    o_ref[0] = o.astype(jnp.bfloat16)
    st_ref[0, 0, :] = m[:, 0]
    st_ref[0, 1, :] = lsum[:, 0]


def _merge_body(
    o_ref,
    st_ref,
    wo_ref,
    out_ref,
    comm_o,
    comm_st,
    acc_o,
    acc_st,
    ctx,
    send_o_sem,
    recv_o_sem,
    send_st_sem,
    recv_st_sem,
):
    my = lax.axis_index("i")
    partners = [my ^ 1, my ^ 2]

    barrier_sem = pltpu.get_barrier_semaphore()
    for p in partners:
        pl.semaphore_signal(
            barrier_sem, inc=1, device_id=(p,),
            device_id_type=pl.DeviceIdType.MESH,
        )
    pl.semaphore_wait(barrier_sem, 2)

    for r in range(2):
        p = partners[r]
        src_o = o_ref if r == 0 else acc_o
        src_st = st_ref if r == 0 else acc_st
        rd_o = pltpu.make_async_remote_copy(
            src_ref=src_o,
            dst_ref=comm_o.at[r],
            send_sem=send_o_sem.at[r],
            recv_sem=recv_o_sem.at[r],
            device_id=(p,),
            device_id_type=pl.DeviceIdType.MESH,
        )
        rd_st = pltpu.make_async_remote_copy(
            src_ref=src_st,
            dst_ref=comm_st.at[r],
            send_sem=send_st_sem.at[r],
            recv_sem=recv_st_sem.at[r],
            device_id=(p,),
            device_id_type=pl.DeviceIdType.MESH,
        )
        rd_o.start()
        rd_st.start()
        rd_o.wait()
        rd_st.wait()

        m_a = src_st[:, 0, :]
        l_a = src_st[:, 1, :]
        m_b = comm_st[r, :, 0, :]
        l_b = comm_st[r, :, 1, :]
        mx = jnp.maximum(m_a, m_b)
        sa = jnp.exp(m_a - mx)
        sb = jnp.exp(m_b - mx)
        merged = (
            src_o[...].astype(jnp.float32) * sa[:, :, None]
            + comm_o[r].astype(jnp.float32) * sb[:, :, None]
        )
        acc_o[...] = merged.astype(jnp.bfloat16)
        acc_st[:, 0, :] = mx
        acc_st[:, 1, :] = l_a * sa + l_b * sb

    lsum = acc_st[:, 1, :]
    ctxv = acc_o[...].astype(jnp.float32) / lsum[:, :, None]
    for hh in range(HQ):
        ctx[:, hh * DH:(hh + 1) * DH] = ctxv[hh].astype(jnp.bfloat16)
    out_ref[0] = jnp.dot(
        ctx[...],
        wo_ref[...].astype(jnp.bfloat16),
        preferred_element_type=jnp.float32,
    )


def kernel(x, Wq, K_ext, V_ext, Wo):
    o, stats = pl.pallas_call(
        _attn_body,
        grid=(HQ,),
        in_specs=[
            pl.BlockSpec((1, SQ, DM), lambda h: (0, 0, 0)),
            pl.BlockSpec((DM, DH), lambda h: (0, h)),
            pl.BlockSpec(memory_space=pl.ANY),
            pl.BlockSpec(memory_space=pl.ANY),
        ],
        out_shape=[
            jax.ShapeDtypeStruct((HQ, SQ, DH), jnp.bfloat16),
            jax.ShapeDtypeStruct((HQ, 2, SQ), jnp.float32),
        ],
        out_specs=[
            pl.BlockSpec((1, SQ, DH), lambda h: (h, 0, 0)),
            pl.BlockSpec((1, 2, SQ), lambda h: (h, 0, 0)),
        ],
        scratch_shapes=[
            pltpu.VMEM((SQ, SKV_LOCAL), jnp.float32),
            pltpu.VMEM((2, SKV_LOCAL, DH), jnp.float32),
            pltpu.VMEM((2, SKV_LOCAL, DH), jnp.float32),
            pltpu.SemaphoreType.DMA((2, NSPLIT)),
            pltpu.SemaphoreType.DMA((2, NSPLIT)),
        ],
        compiler_params=pltpu.CompilerParams(
            dimension_semantics=("arbitrary",),
        ),
    )(x, Wq, K_ext, V_ext)

    out = pl.pallas_call(
        _merge_body,
        in_specs=[
            pl.BlockSpec(memory_space=pltpu.VMEM),
            pl.BlockSpec(memory_space=pltpu.VMEM),
            pl.BlockSpec(memory_space=pltpu.VMEM),
        ],
        out_shape=jax.ShapeDtypeStruct((1, SQ, DM), jnp.float32),
        out_specs=pl.BlockSpec(memory_space=pltpu.VMEM),
        scratch_shapes=[
            pltpu.VMEM((2, HQ, SQ, DH), jnp.bfloat16),
            pltpu.VMEM((2, HQ, 2, SQ), jnp.float32),
            pltpu.VMEM((HQ, SQ, DH), jnp.bfloat16),
            pltpu.VMEM((HQ, 2, SQ), jnp.float32),
            pltpu.VMEM((SQ, DM), jnp.bfloat16),
            pltpu.SemaphoreType.DMA((2,)),
            pltpu.SemaphoreType.DMA((2,)),
            pltpu.SemaphoreType.DMA((2,)),
            pltpu.SemaphoreType.DMA((2,)),
        ],
        compiler_params=pltpu.CompilerParams(collective_id=0),
    )(o, stats, Wo)
    return out


# device time: 45411 ns/iter; 1.5344x vs baseline; 1.5344x over previous
import jax
import jax.numpy as jnp
from jax import lax
from jax.experimental import pallas as pl
from jax.experimental.pallas import tpu as pltpu

N_DEV = 4
SQ = 256
SKV_LOCAL = 4096
HQ = 8
DH = 128
DM = 1024
SCALE = 0.08838834764831843
NEG = -1e9

NSPLIT = 4
CHUNK = SKV_LOCAL // NSPLIT


def _head_copies(k_hbm, v_hbm, kbuf, vbuf, ksem, vsem, head, slot):
    copies = []
    for c in range(NSPLIT):
        sl = pl.ds(c * CHUNK, CHUNK)
        copies.append(pltpu.make_async_copy(
            k_hbm.at[0, sl, head, :], kbuf.at[slot, sl, :], ksem.at[slot, c]
        ))
        copies.append(pltpu.make_async_copy(
            v_hbm.at[0, sl, head, :], vbuf.at[slot, sl, :], vsem.at[slot, c]
        ))
    return copies


def _body(
    x_ref, wq_ref, wo_ref, k_hbm, v_hbm, out_ref,
    bias_ref, kbuf, vbuf, ksem, vsem,
    obuf, stbuf, macc, macc_st, comm_o, comm_st, ctx,
    so, ro, ss, rs,
):
    h = pl.program_id(0)
    slot = h % 2
    nslot = (h + 1) % 2
    my = lax.axis_index("i")
    p0 = my ^ 1
    p1 = my ^ 2

    def rd_o(r, j, src, partner):
        return pltpu.make_async_remote_copy(
            src_ref=src.at[j],
            dst_ref=comm_o.at[r, j],
            send_sem=so.at[r, j],
            recv_sem=ro.at[r, j],
            device_id=(partner,),
            device_id_type=pl.DeviceIdType.MESH,
        )

    def rd_st(r, j, src, partner):
        return pltpu.make_async_remote_copy(
            src_ref=src.at[j],
            dst_ref=comm_st.at[r, j],
            send_sem=ss.at[r, j],
            recv_sem=rs.at[r, j],
            device_id=(partner,),
            device_id_type=pl.DeviceIdType.MESH,
        )

    @pl.when(h == 0)
    def _():
        barrier_sem = pltpu.get_barrier_semaphore()
        for p in (p0, p1):
            pl.semaphore_signal(
                barrier_sem, inc=1, device_id=(p,),
                device_id_type=pl.DeviceIdType.MESH,
            )
        pl.semaphore_wait(barrier_sem, 2)
        for cp in _head_copies(k_hbm, v_hbm, kbuf, vbuf, ksem, vsem, 0, 0):
            cp.start()
        qb = lax.broadcasted_iota(jnp.int32, (SQ, SKV_LOCAL), 0) // 64
        kb = lax.broadcasted_iota(jnp.int32, (SQ, SKV_LOCAL), 1) // 64 + my * 64
        mask = (qb == kb) | (kb == 0) | (((qb + kb) % 3) == 0)
        bias_ref[...] = jnp.where(mask, 0.0, NEG).astype(jnp.float32)

    @pl.when(h + 1 < HQ)
    def _():
        for cp in _head_copies(
            k_hbm, v_hbm, kbuf, vbuf, ksem, vsem, h + 1, nslot
        ):
            cp.start()

    q = jnp.dot(
        x_ref[0].astype(jnp.bfloat16),
        wq_ref[...].astype(jnp.bfloat16),
        preferred_element_type=jnp.float32,
    )
    copies = _head_copies(k_hbm, v_hbm, kbuf, vbuf, ksem, vsem, h, slot)
    for cp in copies[0::2]:
        cp.wait()
    s = lax.dot_general(
        q.astype(jnp.bfloat16),
        kbuf[slot].astype(jnp.bfloat16),
        ((((1,), (1,))), ((), ())),
        preferred_element_type=jnp.float32,
    )
    s = s * SCALE + bias_ref[...]
    m = jnp.max(s, axis=1, keepdims=True)
    w = jnp.exp(s - m)
    lsum = jnp.sum(w, axis=1, keepdims=True)
    for cp in copies[1::2]:
        cp.wait()
    o = lax.dot_general(
        w.astype(jnp.bfloat16),
        vbuf[slot].astype(jnp.bfloat16),
        ((((1,), (0,))), ((), ())),
        preferred_element_type=jnp.float32,
    )
    obuf[h] = o.astype(jnp.bfloat16)
    stbuf[h, 0, :] = m[:, 0]
    stbuf[h, 1, :] = lsum[:, 0]

    rd_o(0, h, obuf, p0).start()
    rd_st(0, h, stbuf, p0).start()

    def merge0_send1(j):
        rd_o(0, j, obuf, p0).wait_recv()
        rd_st(0, j, stbuf, p0).wait_recv()
        m_a = stbuf[j, 0, :]
        l_a = stbuf[j, 1, :]
        m_b = comm_st[0, j, 0, :]
        l_b = comm_st[0, j, 1, :]
        mx = jnp.maximum(m_a, m_b)
        sa = jnp.exp(m_a - mx)
        sb = jnp.exp(m_b - mx)
        merged = (
            obuf[j].astype(jnp.float32) * sa[:, None]
            + comm_o[0, j].astype(jnp.float32) * sb[:, None]
        )
        macc[j] = merged.astype(jnp.bfloat16)
        macc_st[j, 0, :] = mx
        macc_st[j, 1, :] = l_a * sa + l_b * sb
        rd_o(1, j, macc, p1).start()
        rd_st(1, j, macc_st, p1).start()

    @pl.when(h > 0)
    def _():
        merge0_send1(h - 1)

    @pl.when(h == HQ - 1)
    def _():
        merge0_send1(HQ - 1)
        for j in range(HQ):
            rd_o(1, j, macc, p1).wait_recv()
            rd_st(1, j, macc_st, p1).wait_recv()
            m_a = macc_st[j, 0, :]
            l_a = macc_st[j, 1, :]
            m_b = comm_st[1, j, 0, :]
            l_b = comm_st[1, j, 1, :]
            mx = jnp.maximum(m_a, m_b)
            sa = jnp.exp(m_a - mx)
            sb = jnp.exp(m_b - mx)
            of = (
                macc[j].astype(jnp.float32) * sa[:, None]
                + comm_o[1, j].astype(jnp.float32) * sb[:, None]
            )
            lf = l_a * sa + l_b * sb
            ctx[:, j * DH:(j + 1) * DH] = (of / lf[:, None]).astype(
                jnp.bfloat16
            )
        for j in range(HQ):
            rd_o(0, j, obuf, p0).wait_send()
            rd_st(0, j, stbuf, p0).wait_send()
            rd_o(1, j, macc, p1).wait_send()
            rd_st(1, j, macc_st, p1).wait_send()
        out_ref[0] = jnp.dot(
            ctx[...],
            wo_ref[...].astype(jnp.bfloat16),
            preferred_element_type=jnp.float32,
        )


def kernel(x, Wq, K_ext, V_ext, Wo):
    return pl.pallas_call(
        _body,
        grid=(HQ,),
        in_specs=[
            pl.BlockSpec((1, SQ, DM), lambda h: (0, 0, 0)),
            pl.BlockSpec((DM, DH), lambda h: (0, h)),
            pl.BlockSpec((DM, DM), lambda h: (0, 0)),
            pl.BlockSpec(memory_space=pl.ANY),
            pl.BlockSpec(memory_space=pl.ANY),
        ],
        out_shape=jax.ShapeDtypeStruct((1, SQ, DM), jnp.float32),
        out_specs=pl.BlockSpec((1, SQ, DM), lambda h: (0, 0, 0)),
        scratch_shapes=[
            pltpu.VMEM((SQ, SKV_LOCAL), jnp.float32),
            pltpu.VMEM((2, SKV_LOCAL, DH), jnp.float32),
            pltpu.VMEM((2, SKV_LOCAL, DH), jnp.float32),
            pltpu.SemaphoreType.DMA((2, NSPLIT)),
            pltpu.SemaphoreType.DMA((2, NSPLIT)),
            pltpu.VMEM((HQ, SQ, DH), jnp.bfloat16),
            pltpu.VMEM((HQ, 2, SQ), jnp.float32),
            pltpu.VMEM((HQ, SQ, DH), jnp.bfloat16),
            pltpu.VMEM((HQ, 2, SQ), jnp.float32),
            pltpu.VMEM((2, HQ, SQ, DH), jnp.bfloat16),
            pltpu.VMEM((2, HQ, 2, SQ), jnp.float32),
            pltpu.VMEM((SQ, DM), jnp.bfloat16),
            pltpu.SemaphoreType.DMA((2, HQ)),
            pltpu.SemaphoreType.DMA((2, HQ)),
            pltpu.SemaphoreType.DMA((2, HQ)),
            pltpu.SemaphoreType.DMA((2, HQ)),
        ],
        compiler_params=pltpu.CompilerParams(
            dimension_semantics=("arbitrary",),
            collective_id=0,
        ),
    )(x, Wq, Wo, K_ext, V_ext)
